# baseline (speedup 1.0000x reference)
import functools

import jax
import jax.numpy as jnp
from jax import lax
from jax.experimental import pallas as pl
from jax.experimental.pallas import tpu as pltpu
from jax.experimental.pallas import tpu_sc as plsc

_N = 10000
_D = 128
_K = 1024
_LAMBDA = 0.5
_RB = 400
_NTOT = _N * _N
_W = 20000
_NWIN = _NTOT // _W
_NTILE = 32
_TMAX = (_NWIN + _NTILE - 1) // _NTILE
_VPW = _W // 16
_SUB = 2000
_NSUB = _W // _SUB
_VPS = _SUB // 16
_BUF = 4608
_NBKT = 65536

_sc_mesh = plsc.VectorSubcoreMesh(core_axis_name="c", subcore_axis_name="s")
_sc_params = pltpu.CompilerParams(needs_layout_passes=False)


def _matmul_body(z1_ref, z2_ref, s_ref):
    s_ref[...] = lax.dot_general(
        z1_ref[...], z2_ref[...], (((1,), (1,)), ((), ())),
        preferred_element_type=jnp.float32)


def _similarity(z1, z2):
    return pl.pallas_call(
        _matmul_body,
        grid=(_N // _RB,),
        in_specs=[pl.BlockSpec((_RB, _D), lambda i: (i, 0)),
                  pl.BlockSpec((_N, _D), lambda i: (0, 0))],
        out_specs=pl.BlockSpec((_RB, _N), lambda i: (i, 0)),
        out_shape=jax.ShapeDtypeStruct((_N, _N), jnp.float32),
    )(z1, z2)


def _wid():
    return lax.axis_index("s") * 2 + lax.axis_index("c")


def _bucket_of(v):
    b = lax.bitcast_convert_type(v, jnp.int32)
    key = b ^ (jnp.int32(0x7FFFFFFF) & (b >> 31))
    return lax.shift_right_logical(key, 16) ^ jnp.int32(0x8000)


@functools.partial(
    pl.kernel, mesh=_sc_mesh, compiler_params=_sc_params,
    out_type=jax.ShapeDtypeStruct((_NTILE, _NBKT), jnp.float32),
    scratch_types=[pltpu.VMEM((_NBKT,), jnp.float32),
                   pltpu.VMEM((_W,), jnp.float32)],
)
def _sweep_hist(s_hbm, out_hbm, hist, win):
    w = _wid()

    def init(j, c):
        hist[pl.ds(j * 16, 16)] = jnp.zeros((16,), jnp.float32)
        return c
    lax.fori_loop(0, _NBKT // 16, init, 0)

    ones = jnp.ones((16,), jnp.float32)

    def wbody(t, c):
        widx = w + t * _NTILE

        @pl.when(widx < _NWIN)
        def _():
            pltpu.sync_copy(s_hbm.at[pl.ds(widx * _W, _W)], win)

            def vbody(i, c2):
                for u in range(10):
                    v = win[pl.ds((i * 10 + u) * 16, 16)]
                    bkt = _bucket_of(v)
                    cnt, last = plsc.scan_count(bkt)
                    plsc.addupdate_scatter(hist, [bkt], cnt.astype(jnp.float32),
                                           mask=last)
                return c2
            lax.fori_loop(0, _VPW // 10, vbody, 0)
        return c

    lax.fori_loop(0, _TMAX, wbody, 0)
    pltpu.sync_copy(hist, out_hbm.at[w])


def _boundary_body(h_ref, bhi_ref, blo_ref):
    h = jnp.sum(h_ref[...].astype(jnp.int32), axis=0, keepdims=True)
    iota = lax.broadcasted_iota(jnp.int32, (1, _NBKT), 1)
    kk = jnp.int32(_K)

    def hi_body(i, b):
        trial = b | (jnp.int32(1) << (15 - i))
        cnt = jnp.sum(jnp.where(iota >= trial, h, 0))
        return jnp.where(cnt >= kk, trial, b)
    bhi = lax.fori_loop(0, 16, hi_body, jnp.int32(0))

    def lo_body(i, b):
        cand = b | ((jnp.int32(1) << (15 - i)) - 1)
        cnt = jnp.sum(jnp.where(iota <= cand, h, 0))
        return jnp.where(cnt >= kk, b, b | (jnp.int32(1) << (15 - i)))
    blo = lax.fori_loop(0, 16, lo_body, jnp.int32(0))

    bhi_ref[0] = bhi
    blo_ref[0] = blo


def _boundaries(hists):
    return pl.pallas_call(
        _boundary_body,
        out_specs=(pl.BlockSpec(memory_space=pltpu.SMEM),
                   pl.BlockSpec(memory_space=pltpu.SMEM)),
        out_shape=(jax.ShapeDtypeStruct((1,), jnp.int32),
                   jax.ShapeDtypeStruct((1,), jnp.int32)),
    )(hists)


@functools.partial(
    pl.kernel, mesh=_sc_mesh, compiler_params=_sc_params,
    out_type=(jax.ShapeDtypeStruct((_NTILE, _BUF), jnp.float32),
              jax.ShapeDtypeStruct((_NTILE, _BUF), jnp.float32)),
    scratch_types=[pltpu.VMEM((_W,), jnp.float32),
                   pltpu.VMEM((_BUF,), jnp.float32),
                   pltpu.VMEM((_BUF,), jnp.float32),
                   pltpu.VMEM((16,), jnp.int32),
                   pltpu.VMEM((16,), jnp.int32),
                   pltpu.SMEM((2,), jnp.int32)],
)
def _sweep_collect(s_hbm, bhi_hbm, blo_hbm, outh_hbm, outl_hbm,
                   win, bufh, bufl, bhi_v, blo_v, cnts):
    w = _wid()
    pltpu.sync_copy(bhi_hbm, bhi_v)
    pltpu.sync_copy(blo_hbm, blo_v)
    bhi = bhi_v[...]
    blo = blo_v[...]

    def inith(j, c):
        bufh[pl.ds(j * 16, 16)] = jnp.full((16,), -jnp.inf, jnp.float32)
        bufl[pl.ds(j * 16, 16)] = jnp.full((16,), jnp.inf, jnp.float32)
        return c
    lax.fori_loop(0, _BUF // 16, inith, 0)
    cnts[0] = jnp.int32(0)
    cnts[1] = jnp.int32(0)

    one16 = jnp.ones((16,), jnp.int32)
    zero16 = jnp.zeros((16,), jnp.int32)

    def wbody(t, c):
        widx = w + t * _NTILE

        @pl.when(widx < _NWIN)
        def _():
            pltpu.sync_copy(s_hbm.at[pl.ds(widx * _W, _W)], win)

            def sbody(s, c2):
                base = s * _SUB

                def cbody(r, accs):
                    ah, al = accs
                    for u in range(5):
                        bkt = _bucket_of(win[pl.ds(base + (r * 5 + u) * 16, 16)])
                        ah = ah + jnp.where(bkt >= bhi, one16, zero16)
                        al = al + jnp.where(bkt <= blo, one16, zero16)
                    return ah, al
                ah, al = lax.fori_loop(0, _VPS // 5, cbody, (zero16, zero16))
                ch = jnp.sum(ah)
                cl = jnp.sum(al)

                @pl.when(ch > 0)
                def _():
                    def hbody(r, c3):
                        v = win[pl.ds(base + r * 16, 16)]
                        m = _bucket_of(v) >= bhi
                        cnt = cnts[0]
                        cc = jnp.minimum(cnt, jnp.int32(_BUF - 16))
                        plsc.store_compressed(bufh.at[pl.ds(cc, 16)], v, mask=m)
                        pc = jnp.max(plsc.all_reduce_population_count(m))
                        cnts[0] = cnt + pc
                        return c3
                    lax.fori_loop(0, _VPS, hbody, 0)

                @pl.when(cl > 0)
                def _():
                    def lbody(r, c3):
                        v = win[pl.ds(base + r * 16, 16)]
                        m = _bucket_of(v) <= blo
                        cnt = cnts[1]
                        cc = jnp.minimum(cnt, jnp.int32(_BUF - 16))
                        plsc.store_compressed(bufl.at[pl.ds(cc, 16)], v, mask=m)
                        pc = jnp.max(plsc.all_reduce_population_count(m))
                        cnts[1] = cnt + pc
                        return c3
                    lax.fori_loop(0, _VPS, lbody, 0)
                return c2
            lax.fori_loop(0, _NSUB, sbody, 0)
        return c

    lax.fori_loop(0, _TMAX, wbody, 0)
    pltpu.sync_copy(bufh, outh_hbm.at[w])
    pltpu.sync_copy(bufl, outl_hbm.at[w])


def _final_body(ch_ref, cl_ref, z1_ref, z2_ref, out_ref):
    ch = ch_ref[...]
    cl = cl_ref[...]
    z1 = z1_ref[...]
    z2 = z2_ref[...]
    kk = jnp.int32(_K)

    def f_of_u(u):
        kv = jnp.full(ch.shape, u, jnp.int32) ^ jnp.int32(-2147483648)
        bv = jnp.where(kv >= 0, kv, kv ^ jnp.int32(0x7FFFFFFF))
        return lax.bitcast_convert_type(bv, jnp.float32)

    def hi_body(i, u):
        trial = u | (jnp.int32(1) << (31 - i))
        cnt = jnp.sum((ch >= f_of_u(trial)).astype(jnp.int32))
        return jnp.where(cnt >= kk, trial, u)
    uhi = lax.fori_loop(0, 32, hi_body, jnp.int32(0))
    tmat = f_of_u(uhi)
    v_t = jnp.max(tmat)
    sp_vt = jnp.max(jnp.logaddexp(0.0, tmat))
    gt = ch > tmat
    c_gt = jnp.sum(gt.astype(jnp.int32))
    sp_gt = jnp.sum(jnp.where(gt, jnp.logaddexp(0.0, ch), 0.0))
    x_gt = jnp.sum(jnp.where(gt, ch, 0.0))
    rem = (kk - c_gt).astype(jnp.float32)
    sum_sp_hi = sp_gt + rem * sp_vt
    sum_x_hi = x_gt + rem * v_t

    def f_of_u_l(u):
        kv = jnp.full(cl.shape, u, jnp.int32) ^ jnp.int32(-2147483648)
        bv = jnp.where(kv >= 0, kv, kv ^ jnp.int32(0x7FFFFFFF))
        return lax.bitcast_convert_type(bv, jnp.float32)

    def lo_body(i, u):
        cand = u | ((jnp.int32(1) << (31 - i)) - 1)
        cnt = jnp.sum((cl <= f_of_u_l(cand)).astype(jnp.int32))
        return jnp.where(cnt >= kk, u, u | (jnp.int32(1) << (31 - i)))
    ulo = lax.fori_loop(0, 32, lo_body, jnp.int32(0))
    tmat_l = f_of_u_l(ulo)
    v_b = jnp.max(tmat_l)
    sp_vb = jnp.max(jnp.logaddexp(0.0, tmat_l))
    lt = cl < tmat_l
    c_lt = jnp.sum(lt.astype(jnp.int32))
    sp_lt = jnp.sum(jnp.where(lt, jnp.logaddexp(0.0, cl), 0.0))
    rem_l = (kk - c_lt).astype(jnp.float32)
    sum_sp_lo = sp_lt + rem_l * sp_vb

    diag = jnp.sum(z1 * z2, axis=1, keepdims=True)
    sum_sp_diag = jnp.sum(jnp.logaddexp(0.0, diag))
    sum_diag = jnp.sum(diag)
    mse = jnp.mean((z1 - z2) ** 2)

    total_sp = sum_sp_hi + sum_sp_lo + sum_sp_diag
    pos_x = sum_x_hi + sum_diag
    loss = (total_sp - pos_x) / jnp.float32(_N + 2 * _K)
    loss = loss + jnp.float32(_LAMBDA) * mse * jnp.float32(_N)
    out_ref[0] = loss


def _finalize(cand_hi, cand_lo, z1, z2):
    return pl.pallas_call(
        _final_body,
        out_specs=pl.BlockSpec(memory_space=pltpu.SMEM),
        out_shape=jax.ShapeDtypeStruct((1,), jnp.float32),
    )(cand_hi, cand_lo, z1, z2)


def kernel(epoch, z1, z2):
    s = _similarity(z1, z2)
    s_flat = s.reshape(_NTOT)
    hists = _sweep_hist(s_flat)
    bhi, blo = _boundaries(hists)
    bhi16 = jnp.full((16,), bhi[0], jnp.int32)
    blo16 = jnp.full((16,), blo[0], jnp.int32)
    cand_hi, cand_lo = _sweep_collect(s_flat, bhi16, blo16)
    loss = _finalize(cand_hi, cand_lo, z1, z2)
    return loss[0]


# speedup vs baseline: 1.6782x; 1.6782x over previous
import functools

import jax
import jax.numpy as jnp
from jax import lax
from jax.experimental import pallas as pl
from jax.experimental.pallas import tpu as pltpu
from jax.experimental.pallas import tpu_sc as plsc

_N = 10000
_D = 128
_K = 1024
_LAMBDA = 0.5
_RB = 400
_NTOT = _N * _N
_W = 20000
_NWIN = _NTOT // _W
_NTILE = 32
_TMAX = (_NWIN + _NTILE - 1) // _NTILE
_VPW = _W // 16
_SUB = 2000
_NSUB = _W // _SUB
_VPS = _SUB // 16
_BUF = 4608
_NBKT = 65536

_sc_mesh = plsc.VectorSubcoreMesh(core_axis_name="c", subcore_axis_name="s")
_sc_params = pltpu.CompilerParams(needs_layout_passes=False)


def _matmul_body(z1_ref, z2_ref, s_ref):
    s_ref[...] = lax.dot_general(
        z1_ref[...], z2_ref[...], (((1,), (1,)), ((), ())),
        preferred_element_type=jnp.float32)


def _similarity(z1, z2):
    return pl.pallas_call(
        _matmul_body,
        grid=(_N // _RB,),
        in_specs=[pl.BlockSpec((_RB, _D), lambda i: (i, 0)),
                  pl.BlockSpec((_N, _D), lambda i: (0, 0))],
        out_specs=pl.BlockSpec((_RB, _N), lambda i: (i, 0)),
        out_shape=jax.ShapeDtypeStruct((_N, _N), jnp.float32),
    )(z1, z2)


def _wid():
    return lax.axis_index("s") * 2 + lax.axis_index("c")


def _bucket_of(v):
    b = lax.bitcast_convert_type(v, jnp.int32)
    key = b ^ (jnp.int32(0x7FFFFFFF) & (b >> 31))
    return lax.shift_right_logical(key, 16) ^ jnp.int32(0x8000)


_WSLOT = 176


@functools.partial(
    pl.kernel, mesh=_sc_mesh, compiler_params=_sc_params,
    out_type=(jax.ShapeDtypeStruct((_NTILE, _NBKT), jnp.float32),
              jax.ShapeDtypeStruct((_NTILE, _WSLOT), jnp.float32),
              jax.ShapeDtypeStruct((_NTILE, _WSLOT), jnp.float32)),
    scratch_types=[pltpu.VMEM((_NBKT,), jnp.float32),
                   pltpu.VMEM((_W,), jnp.float32),
                   pltpu.VMEM((_WSLOT,), jnp.float32),
                   pltpu.VMEM((_WSLOT,), jnp.float32)],
)
def _sweep_hist(s_hbm, out_hbm, wmax_hbm, wmin_hbm, hist, win, wmax, wmin):
    w = _wid()

    def init(j, c):
        hist[pl.ds(j * 16, 16)] = jnp.zeros((16,), jnp.float32)
        return c
    lax.fori_loop(0, _NBKT // 16, init, 0)

    def initw(j, c):
        wmax[pl.ds(j * 16, 16)] = jnp.full((16,), -jnp.inf, jnp.float32)
        wmin[pl.ds(j * 16, 16)] = jnp.full((16,), jnp.inf, jnp.float32)
        return c
    lax.fori_loop(0, _WSLOT // 16, initw, 0)

    ones = jnp.ones((16,), jnp.float32)
    lane0 = lax.iota(jnp.int32, 16) == 0

    def wbody(t, c):
        widx = w + t * _NTILE

        @pl.when(widx < _NWIN)
        def _():
            pltpu.sync_copy(s_hbm.at[pl.ds(widx * _W, _W)], win)

            def vbody(i, mm):
                amax, amin = mm
                for u in range(10):
                    v = win[pl.ds((i * 10 + u) * 16, 16)]
                    plsc.addupdate_scatter(hist, [_bucket_of(v)], ones)
                    amax = jnp.maximum(amax, v)
                    amin = jnp.minimum(amin, v)
                return amax, amin
            ninf = jnp.full((16,), -jnp.inf, jnp.float32)
            amax, amin = lax.fori_loop(0, _VPW // 10, vbody, (ninf, -ninf))
            smax = jnp.full((16,), jnp.max(amax), jnp.float32)
            smin = jnp.full((16,), jnp.min(amin), jnp.float32)
            plsc.store_compressed(wmax.at[pl.ds(t, 16)], smax, mask=lane0)
            plsc.store_compressed(wmin.at[pl.ds(t, 16)], smin, mask=lane0)
        return c

    lax.fori_loop(0, _TMAX, wbody, 0)
    pltpu.sync_copy(hist, out_hbm.at[w])
    pltpu.sync_copy(wmax, wmax_hbm.at[w])
    pltpu.sync_copy(wmin, wmin_hbm.at[w])


def _boundary_body(h_ref, wmax_ref, wmin_ref, bhi_ref, blo_ref, flags_ref):
    h = jnp.sum(h_ref[...].astype(jnp.int32), axis=0, keepdims=True)
    iota = lax.broadcasted_iota(jnp.int32, (1, _NBKT), 1)
    kk = jnp.int32(_K)

    def hi_body(i, b):
        trial = b | (jnp.int32(1) << (15 - i))
        cnt = jnp.sum(jnp.where(iota >= trial, h, 0))
        return jnp.where(cnt >= kk, trial, b)
    bhi = lax.fori_loop(0, 16, hi_body, jnp.int32(0))

    def lo_body(i, b):
        cand = b | ((jnp.int32(1) << (15 - i)) - 1)
        cnt = jnp.sum(jnp.where(iota <= cand, h, 0))
        return jnp.where(cnt >= kk, b, b | (jnp.int32(1) << (15 - i)))
    blo = lax.fori_loop(0, 16, lo_body, jnp.int32(0))

    bhi_ref[0] = bhi
    blo_ref[0] = blo

    def edge_f(u):
        kv = jnp.full(wmax_ref.shape, u, jnp.int32) ^ jnp.int32(-2147483648)
        bv = jnp.where(kv >= 0, kv, kv ^ jnp.int32(0x7FFFFFFF))
        return lax.bitcast_convert_type(bv, jnp.float32)

    f_hi = edge_f(bhi << 16)
    f_lo = edge_f((blo << 16) | jnp.int32(0xFFFF))
    need = (wmax_ref[...] >= f_hi) | (wmin_ref[...] <= f_lo)
    flags_ref[...] = jnp.where(need, 1, 0).astype(jnp.int32)


def _boundaries(hists, wmax, wmin):
    return pl.pallas_call(
        _boundary_body,
        out_specs=(pl.BlockSpec(memory_space=pltpu.SMEM),
                   pl.BlockSpec(memory_space=pltpu.SMEM),
                   pl.BlockSpec((_NTILE, _WSLOT), lambda: (0, 0))),
        out_shape=(jax.ShapeDtypeStruct((1,), jnp.int32),
                   jax.ShapeDtypeStruct((1,), jnp.int32),
                   jax.ShapeDtypeStruct((_NTILE, _WSLOT), jnp.int32)),
    )(hists, wmax, wmin)


@functools.partial(
    pl.kernel, mesh=_sc_mesh, compiler_params=_sc_params,
    out_type=(jax.ShapeDtypeStruct((_NTILE, _BUF), jnp.float32),
              jax.ShapeDtypeStruct((_NTILE, _BUF), jnp.float32)),
    scratch_types=[pltpu.VMEM((_W,), jnp.float32),
                   pltpu.VMEM((_BUF,), jnp.float32),
                   pltpu.VMEM((_BUF,), jnp.float32),
                   pltpu.VMEM((16,), jnp.int32),
                   pltpu.VMEM((16,), jnp.int32),
                   pltpu.VMEM((_WSLOT,), jnp.int32),
                   pltpu.SMEM((2,), jnp.int32)],
)
def _sweep_collect(s_hbm, bhi_hbm, blo_hbm, flags_hbm, outh_hbm, outl_hbm,
                   win, bufh, bufl, bhi_v, blo_v, flags_v, cnts):
    w = _wid()
    pltpu.sync_copy(bhi_hbm, bhi_v)
    pltpu.sync_copy(blo_hbm, blo_v)
    pltpu.sync_copy(flags_hbm.at[w], flags_v)
    bhi = bhi_v[...]
    blo = blo_v[...]

    def inith(j, c):
        bufh[pl.ds(j * 16, 16)] = jnp.full((16,), -jnp.inf, jnp.float32)
        bufl[pl.ds(j * 16, 16)] = jnp.full((16,), jnp.inf, jnp.float32)
        return c
    lax.fori_loop(0, _BUF // 16, inith, 0)
    cnts[0] = jnp.int32(0)
    cnts[1] = jnp.int32(0)

    one16 = jnp.ones((16,), jnp.int32)
    zero16 = jnp.zeros((16,), jnp.int32)

    def wbody(t, c):
        widx = w + t * _NTILE
        flag = flags_v[pl.ds(t, 16)][0]

        @pl.when((widx < _NWIN) & (flag > 0))
        def _():
            pltpu.sync_copy(s_hbm.at[pl.ds(widx * _W, _W)], win)

            def sbody(s, c2):
                base = s * _SUB

                def cbody(r, accs):
                    ah, al = accs
                    for u in range(5):
                        bkt = _bucket_of(win[pl.ds(base + (r * 5 + u) * 16, 16)])
                        ah = ah + jnp.where(bkt >= bhi, one16, zero16)
                        al = al + jnp.where(bkt <= blo, one16, zero16)
                    return ah, al
                ah, al = lax.fori_loop(0, _VPS // 5, cbody, (zero16, zero16))
                ch = jnp.sum(ah)
                cl = jnp.sum(al)

                @pl.when(ch > 0)
                def _():
                    def hbody(r, c3):
                        v = win[pl.ds(base + r * 16, 16)]
                        m = _bucket_of(v) >= bhi
                        cnt = cnts[0]
                        cc = jnp.minimum(cnt, jnp.int32(_BUF - 16))
                        plsc.store_compressed(bufh.at[pl.ds(cc, 16)], v, mask=m)
                        pc = jnp.max(plsc.all_reduce_population_count(m))
                        cnts[0] = cnt + pc
                        return c3
                    lax.fori_loop(0, _VPS, hbody, 0)

                @pl.when(cl > 0)
                def _():
                    def lbody(r, c3):
                        v = win[pl.ds(base + r * 16, 16)]
                        m = _bucket_of(v) <= blo
                        cnt = cnts[1]
                        cc = jnp.minimum(cnt, jnp.int32(_BUF - 16))
                        plsc.store_compressed(bufl.at[pl.ds(cc, 16)], v, mask=m)
                        pc = jnp.max(plsc.all_reduce_population_count(m))
                        cnts[1] = cnt + pc
                        return c3
                    lax.fori_loop(0, _VPS, lbody, 0)
                return c2
            lax.fori_loop(0, _NSUB, sbody, 0)
        return c

    lax.fori_loop(0, _TMAX, wbody, 0)
    pltpu.sync_copy(bufh, outh_hbm.at[w])
    pltpu.sync_copy(bufl, outl_hbm.at[w])


def _final_body(ch_ref, cl_ref, z1_ref, z2_ref, out_ref):
    ch = ch_ref[...]
    cl = cl_ref[...]
    z1 = z1_ref[...]
    z2 = z2_ref[...]
    kk = jnp.int32(_K)

    def f_of_u(u):
        kv = jnp.full(ch.shape, u, jnp.int32) ^ jnp.int32(-2147483648)
        bv = jnp.where(kv >= 0, kv, kv ^ jnp.int32(0x7FFFFFFF))
        return lax.bitcast_convert_type(bv, jnp.float32)

    def hi_body(i, u):
        trial = u | (jnp.int32(1) << (31 - i))
        cnt = jnp.sum((ch >= f_of_u(trial)).astype(jnp.int32))
        return jnp.where(cnt >= kk, trial, u)
    uhi = lax.fori_loop(0, 32, hi_body, jnp.int32(0))
    tmat = f_of_u(uhi)
    v_t = jnp.max(tmat)
    sp_vt = jnp.max(jnp.logaddexp(0.0, tmat))
    gt = ch > tmat
    c_gt = jnp.sum(gt.astype(jnp.int32))
    sp_gt = jnp.sum(jnp.where(gt, jnp.logaddexp(0.0, ch), 0.0))
    x_gt = jnp.sum(jnp.where(gt, ch, 0.0))
    rem = (kk - c_gt).astype(jnp.float32)
    sum_sp_hi = sp_gt + rem * sp_vt
    sum_x_hi = x_gt + rem * v_t

    def f_of_u_l(u):
        kv = jnp.full(cl.shape, u, jnp.int32) ^ jnp.int32(-2147483648)
        bv = jnp.where(kv >= 0, kv, kv ^ jnp.int32(0x7FFFFFFF))
        return lax.bitcast_convert_type(bv, jnp.float32)

    def lo_body(i, u):
        cand = u | ((jnp.int32(1) << (31 - i)) - 1)
        cnt = jnp.sum((cl <= f_of_u_l(cand)).astype(jnp.int32))
        return jnp.where(cnt >= kk, u, u | (jnp.int32(1) << (31 - i)))
    ulo = lax.fori_loop(0, 32, lo_body, jnp.int32(0))
    tmat_l = f_of_u_l(ulo)
    v_b = jnp.max(tmat_l)
    sp_vb = jnp.max(jnp.logaddexp(0.0, tmat_l))
    lt = cl < tmat_l
    c_lt = jnp.sum(lt.astype(jnp.int32))
    sp_lt = jnp.sum(jnp.where(lt, jnp.logaddexp(0.0, cl), 0.0))
    rem_l = (kk - c_lt).astype(jnp.float32)
    sum_sp_lo = sp_lt + rem_l * sp_vb

    diag = jnp.sum(z1 * z2, axis=1, keepdims=True)
    sum_sp_diag = jnp.sum(jnp.logaddexp(0.0, diag))
    sum_diag = jnp.sum(diag)
    mse = jnp.mean((z1 - z2) ** 2)

    total_sp = sum_sp_hi + sum_sp_lo + sum_sp_diag
    pos_x = sum_x_hi + sum_diag
    loss = (total_sp - pos_x) / jnp.float32(_N + 2 * _K)
    loss = loss + jnp.float32(_LAMBDA) * mse * jnp.float32(_N)
    out_ref[0] = loss


def _finalize(cand_hi, cand_lo, z1, z2):
    return pl.pallas_call(
        _final_body,
        out_specs=pl.BlockSpec(memory_space=pltpu.SMEM),
        out_shape=jax.ShapeDtypeStruct((1,), jnp.float32),
    )(cand_hi, cand_lo, z1, z2)


def kernel(epoch, z1, z2):
    s = _similarity(z1, z2)
    s_flat = s.reshape(_NTOT)
    hists, wmax, wmin = _sweep_hist(s_flat)
    bhi, blo, flags = _boundaries(hists, wmax, wmin)
    bhi16 = jnp.full((16,), bhi[0], jnp.int32)
    blo16 = jnp.full((16,), blo[0], jnp.int32)
    cand_hi, cand_lo = _sweep_collect(s_flat, bhi16, blo16, flags)
    loss = _finalize(cand_hi, cand_lo, z1, z2)
    return loss[0]
